# baseline (speedup 1.0000x reference)
import functools

import jax
import jax.numpy as jnp
import numpy as np
from jax import lax
from jax.experimental import pallas as pl
from jax.experimental.pallas import tpu as pltpu
from jax.experimental.pallas import tpu_sc as plsc

N = 10000
C = 128
CH = C // 2
E = 320000
NP = 10240
NC = 2
NS = 16
NT = NC * NS
K = 128
NB = 160
EP = NS * NB * K
ZB = 64
STR = NP // NS
RB = 1280
DR = NP // RB

_mesh = plsc.VectorSubcoreMesh(core_axis_name="c", subcore_axis_name="s")

_EPAD = np.tile((N + (np.arange(EP - E, dtype=np.int32) % (NP - N)))[None, :],
                (2, 1))


NBD = NB // NC


@functools.partial(
    pl.kernel,
    mesh=_mesh,
    compiler_params=pltpu.CompilerParams(use_tc_tiling_on_sc=False),
    out_type=jax.ShapeDtypeStruct((2 * DR, 1, RB), jnp.float32),
    scratch_types=[
        pltpu.VMEM((NBD, K), jnp.int32),
        pltpu.VMEM((K,), jnp.float32),
        pltpu.VMEM((STR,), jnp.float32),
        pltpu.VMEM_SHARED((NP,), jnp.float32),
    ],
)
def _deg_kernel(tar_hbm, out_hbm, tidx_v, ones_v, zer_v, deg_sp):
    c = lax.axis_index("c")
    s = lax.axis_index("s")

    def zf(i, carry):
        zer_v[pl.ds(i * 16, 16)] = jnp.zeros((16,), jnp.float32)
        return carry

    lax.fori_loop(0, STR // 16, zf, 0)

    def of(i, carry):
        ones_v[pl.ds(i * 16, 16)] = jnp.ones((16,), jnp.float32)
        return carry

    lax.fori_loop(0, K // 16, of, 0)

    pltpu.sync_copy(zer_v, deg_sp.at[pl.ds(s * STR, STR)])
    pltpu.sync_copy(tar_hbm.at[s, pl.ds(c * NBD, NBD)], tidx_v)
    plsc.subcore_barrier()

    def body(j, carry):
        pltpu.sync_copy(ones_v, deg_sp.at[tidx_v.at[j]], add=True)
        return carry

    lax.fori_loop(0, NBD, body, 0)
    plsc.subcore_barrier()
    pltpu.sync_copy(deg_sp.at[pl.ds(s * STR, STR)],
                    out_hbm.at[c * DR + s // 2, 0, pl.ds((s % 2) * STR, STR)])


@functools.partial(
    pl.kernel,
    mesh=_mesh,
    compiler_params=pltpu.CompilerParams(use_tc_tiling_on_sc=False),
    out_type=jax.ShapeDtypeStruct((NP, C), jnp.bfloat16),
    scratch_types=[
        pltpu.VMEM((NB, K), jnp.int32),
        pltpu.VMEM((NB, K), jnp.int32),
        pltpu.VMEM((K, CH), jnp.bfloat16),
        pltpu.VMEM((K, CH), jnp.bfloat16),
        pltpu.VMEM((K, CH), jnp.bfloat16),
        pltpu.VMEM((K, CH), jnp.bfloat16),
        pltpu.VMEM((K, CH), jnp.bfloat16),
        pltpu.VMEM((K, CH), jnp.bfloat16),
        pltpu.VMEM((K, CH), jnp.bfloat16),
        pltpu.VMEM((K, CH), jnp.bfloat16),
        pltpu.VMEM_SHARED((NP, CH), jnp.bfloat16),
        pltpu.SemaphoreType.DMA,
        pltpu.SemaphoreType.DMA,
        pltpu.SemaphoreType.DMA,
        pltpu.SemaphoreType.DMA,
        pltpu.SemaphoreType.DMA,
        pltpu.SemaphoreType.DMA,
        pltpu.SemaphoreType.DMA,
        pltpu.SemaphoreType.DMA,
    ],
)
def _agg_kernel(src_hbm, tar_hbm, g_hbm, out_hbm,
                sidx_v, tidx_v, rows0_v, rows1_v, rows2_v, rows3_v,
                rows4_v, rows5_v, rows6_v, rows7_v, acc_sp,
                sem0, sem1, sem2, sem3, sem4, sem5, sem6, sem7):
    c = lax.axis_index("c")
    s = lax.axis_index("s")

    pltpu.sync_copy(g_hbm.at[c, pl.ds(s * STR, STR)],
                    acc_sp.at[pl.ds(s * STR, STR)])
    pltpu.sync_copy(src_hbm.at[s], sidx_v)
    pltpu.sync_copy(tar_hbm.at[s], tidx_v)
    plsc.subcore_barrier()

    ga = g_hbm.at[c]
    bufs = [(rows0_v, sem0), (rows1_v, sem1), (rows2_v, sem2),
            (rows3_v, sem3), (rows4_v, sem4), (rows5_v, sem5),
            (rows6_v, sem6), (rows7_v, sem7)]
    for b, (rv, sm) in enumerate(bufs):
        pltpu.async_copy(ga.at[sidx_v.at[b]], rv, sm)

    NR = len(bufs)

    def body(m, carry):
        j = NR * m
        for b, (rv, sm) in enumerate(bufs):
            pltpu.make_async_copy(ga.at[sidx_v.at[j + b]], rv, sm).wait()
            pltpu.sync_copy(rv, acc_sp.at[tidx_v.at[j + b]], add=True)
            pltpu.async_copy(ga.at[sidx_v.at[j + b + NR]], rv, sm)
        return carry

    lax.fori_loop(0, NB // NR - 1, body, 0)

    j = NB - NR
    for b, (rv, sm) in enumerate(bufs):
        pltpu.make_async_copy(ga.at[sidx_v.at[j + b]], rv, sm).wait()
        pltpu.sync_copy(rv, acc_sp.at[tidx_v.at[j + b]], add=True)
    plsc.subcore_barrier()

    pltpu.sync_copy(acc_sp.at[pl.ds(s * STR, STR)],
                    out_hbm.at[pl.ds(s * STR, STR), pl.ds(c * CH, CH)])


def _h_body(x_ref, w_ref, b_ref, h_ref):
    h_ref[...] = jnp.dot(x_ref[...], w_ref[...],
                         preferred_element_type=jnp.float32) + b_ref[...]


_h_call = pl.pallas_call(
    _h_body,
    grid=(NP // RB,),
    in_specs=[
        pl.BlockSpec((RB, C), lambda i: (i, 0)),
        pl.BlockSpec((C, C), lambda i: (0, 0)),
        pl.BlockSpec((1, C), lambda i: (0, 0)),
    ],
    out_specs=pl.BlockSpec((RB, C), lambda i: (i, 0)),
    out_shape=jax.ShapeDtypeStruct((NP, C), jnp.float32),
)


def _g_body(dpa_ref, dpb_ref, h_ref, g_ref, d_ref):
    deg = dpa_ref[0, 0, :] + dpb_ref[0, 0, :] + 1.0
    dinv = lax.rsqrt(deg)
    g = (dinv[:, None] * h_ref[...]).astype(jnp.bfloat16)
    g_ref[0, :, :] = g[:, :CH]
    g_ref[1, :, :] = g[:, CH:]
    d_ref[...] = dinv[:, None]


_g_call = pl.pallas_call(
    _g_body,
    grid=(NP // RB,),
    in_specs=[
        pl.BlockSpec((1, 1, RB), lambda i: (i, 0, 0)),
        pl.BlockSpec((1, 1, RB), lambda i: (i + DR, 0, 0)),
        pl.BlockSpec((RB, C), lambda i: (i, 0)),
    ],
    out_specs=[
        pl.BlockSpec((2, RB, CH), lambda i: (0, i, 0)),
        pl.BlockSpec((RB, 1), lambda i: (i, 0)),
    ],
    out_shape=[
        jax.ShapeDtypeStruct((NC, NP, CH), jnp.bfloat16),
        jax.ShapeDtypeStruct((NP, 1), jnp.float32),
    ],
)


def _fin_body(d_ref, a_ref, o_ref):
    o_ref[...] = d_ref[...] * a_ref[...].astype(jnp.float32)


RF = 2000

_fin_call = pl.pallas_call(
    _fin_body,
    grid=(N // RF,),
    in_specs=[
        pl.BlockSpec((RF, 1), lambda i: (i, 0)),
        pl.BlockSpec((RF, C), lambda i: (i, 0)),
    ],
    out_specs=pl.BlockSpec((RF, C), lambda i: (i, 0)),
    out_shape=jax.ShapeDtypeStruct((N, C), jnp.float32),
)


def kernel(x, edge_index, W, b):
    ei = jnp.concatenate([edge_index, jnp.asarray(_EPAD)], axis=1)
    tar_r = ei[0].reshape(NS, NB, K)
    src_r = ei[1].reshape(NS, NB, K)

    h = _h_call(x, W, b.reshape(1, C))
    degp = _deg_kernel(tar_r)
    g2, dcol = _g_call(degp, degp, h)
    acc = _agg_kernel(src_r, tar_r, g2)
    return _fin_call(dcol, acc)


# speedup vs baseline: 1.0029x; 1.0029x over previous
import functools

import jax
import jax.numpy as jnp
import numpy as np
from jax import lax
from jax.experimental import pallas as pl
from jax.experimental.pallas import tpu as pltpu
from jax.experimental.pallas import tpu_sc as plsc

N = 10000
C = 128
CH = C // 2
E = 320000
NP = 10240
NC = 2
NS = 16
NT = NC * NS
K = 128
NB = 160
EP = NS * NB * K
ZB = 64
STR = NP // NS
RB = 1280
DR = NP // RB

_mesh = plsc.VectorSubcoreMesh(core_axis_name="c", subcore_axis_name="s")

_EPAD = np.tile((N + (np.arange(EP - E, dtype=np.int32) % (NP - N)))[None, :],
                (2, 1))


NBD = NB // NC


@functools.partial(
    pl.kernel,
    mesh=_mesh,
    compiler_params=pltpu.CompilerParams(use_tc_tiling_on_sc=False),
    out_type=jax.ShapeDtypeStruct((2 * DR, 1, RB), jnp.float32),
    scratch_types=[
        pltpu.VMEM((NBD, K), jnp.int32),
        pltpu.VMEM((K,), jnp.float32),
        pltpu.VMEM((STR,), jnp.float32),
        pltpu.VMEM_SHARED((NP,), jnp.float32),
    ],
)
def _deg_kernel(tar_hbm, out_hbm, tidx_v, ones_v, zer_v, deg_sp):
    c = lax.axis_index("c")
    s = lax.axis_index("s")

    def zf(i, carry):
        zer_v[pl.ds(i * 16, 16)] = jnp.zeros((16,), jnp.float32)
        return carry

    lax.fori_loop(0, STR // 16, zf, 0)

    def of(i, carry):
        ones_v[pl.ds(i * 16, 16)] = jnp.ones((16,), jnp.float32)
        return carry

    lax.fori_loop(0, K // 16, of, 0)

    pltpu.sync_copy(zer_v, deg_sp.at[pl.ds(s * STR, STR)])
    pltpu.sync_copy(tar_hbm.at[s, pl.ds(c * NBD, NBD)], tidx_v)
    plsc.subcore_barrier()

    def body(j, carry):
        pltpu.sync_copy(ones_v, deg_sp.at[tidx_v.at[j]], add=True)
        return carry

    lax.fori_loop(0, NBD, body, 0)
    plsc.subcore_barrier()
    pltpu.sync_copy(deg_sp.at[pl.ds(s * STR, STR)],
                    out_hbm.at[c * DR + s // 2, 0, pl.ds((s % 2) * STR, STR)])


@functools.partial(
    pl.kernel,
    mesh=_mesh,
    compiler_params=pltpu.CompilerParams(use_tc_tiling_on_sc=False),
    out_type=jax.ShapeDtypeStruct((NP, C), jnp.bfloat16),
    scratch_types=[
        pltpu.VMEM((NB, K), jnp.int32),
        pltpu.VMEM((NB, K), jnp.int32),
        pltpu.VMEM((K, CH), jnp.bfloat16),
        pltpu.VMEM((K, CH), jnp.bfloat16),
        pltpu.VMEM((K, CH), jnp.bfloat16),
        pltpu.VMEM((K, CH), jnp.bfloat16),
        pltpu.VMEM((K, CH), jnp.bfloat16),
        pltpu.VMEM((K, CH), jnp.bfloat16),
        pltpu.VMEM((K, CH), jnp.bfloat16),
        pltpu.VMEM((K, CH), jnp.bfloat16),
        pltpu.VMEM_SHARED((NP, CH), jnp.bfloat16),
        pltpu.SemaphoreType.DMA,
        pltpu.SemaphoreType.DMA,
        pltpu.SemaphoreType.DMA,
        pltpu.SemaphoreType.DMA,
        pltpu.SemaphoreType.DMA,
        pltpu.SemaphoreType.DMA,
        pltpu.SemaphoreType.DMA,
        pltpu.SemaphoreType.DMA,
    ],
)
def _agg_kernel(src_hbm, tar_hbm, g_hbm, out_hbm,
                sidx_v, tidx_v, rows0_v, rows1_v, rows2_v, rows3_v,
                rows4_v, rows5_v, rows6_v, rows7_v, acc_sp,
                sem0, sem1, sem2, sem3, sem4, sem5, sem6, sem7):
    c = lax.axis_index("c")
    s = lax.axis_index("s")

    pltpu.sync_copy(g_hbm.at[c, pl.ds(s * STR, STR)],
                    acc_sp.at[pl.ds(s * STR, STR)])
    pltpu.sync_copy(src_hbm.at[s], sidx_v)
    pltpu.sync_copy(tar_hbm.at[s], tidx_v)
    plsc.subcore_barrier()

    ga = g_hbm.at[c]
    bufs = [(rows0_v, sem0), (rows1_v, sem1), (rows2_v, sem2),
            (rows3_v, sem3), (rows4_v, sem4), (rows5_v, sem5),
            (rows6_v, sem6), (rows7_v, sem7)]
    for b, (rv, sm) in enumerate(bufs):
        pltpu.async_copy(ga.at[sidx_v.at[b]], rv, sm)

    NR = len(bufs)

    def body(m, carry):
        j = NR * m
        for b, (rv, sm) in enumerate(bufs):
            pltpu.make_async_copy(ga.at[sidx_v.at[j + b]], rv, sm).wait()
            pltpu.sync_copy(rv, acc_sp.at[tidx_v.at[j + b]], add=True)
            pltpu.async_copy(ga.at[sidx_v.at[j + b + NR]], rv, sm)
        return carry

    lax.fori_loop(0, NB // NR - 1, body, 0)

    j = NB - NR
    for b, (rv, sm) in enumerate(bufs):
        pltpu.make_async_copy(ga.at[sidx_v.at[j + b]], rv, sm).wait()
        pltpu.sync_copy(rv, acc_sp.at[tidx_v.at[j + b]], add=True)
    plsc.subcore_barrier()

    pltpu.sync_copy(acc_sp.at[pl.ds(s * STR, STR)],
                    out_hbm.at[pl.ds(s * STR, STR), pl.ds(c * CH, CH)])


def _g_body(dpa_ref, dpb_ref, x_ref, w_ref, b_ref, g_ref, d_ref):
    h = jnp.dot(x_ref[...], w_ref[...],
                preferred_element_type=jnp.float32) + b_ref[...]
    deg = dpa_ref[0, 0, :] + dpb_ref[0, 0, :] + 1.0
    dinv = lax.rsqrt(deg)
    g = (dinv[:, None] * h).astype(jnp.bfloat16)
    g_ref[0, :, :] = g[:, :CH]
    g_ref[1, :, :] = g[:, CH:]
    d_ref[...] = dinv[:, None]


_g_call = pl.pallas_call(
    _g_body,
    grid=(NP // RB,),
    in_specs=[
        pl.BlockSpec((1, 1, RB), lambda i: (i, 0, 0)),
        pl.BlockSpec((1, 1, RB), lambda i: (i + DR, 0, 0)),
        pl.BlockSpec((RB, C), lambda i: (i, 0)),
        pl.BlockSpec((C, C), lambda i: (0, 0)),
        pl.BlockSpec((1, C), lambda i: (0, 0)),
    ],
    out_specs=[
        pl.BlockSpec((2, RB, CH), lambda i: (0, i, 0)),
        pl.BlockSpec((RB, 1), lambda i: (i, 0)),
    ],
    out_shape=[
        jax.ShapeDtypeStruct((NC, NP, CH), jnp.bfloat16),
        jax.ShapeDtypeStruct((NP, 1), jnp.float32),
    ],
)


def _fin_body(d_ref, a_ref, o_ref):
    o_ref[...] = d_ref[...] * a_ref[...].astype(jnp.float32)


RF = 2000

_fin_call = pl.pallas_call(
    _fin_body,
    grid=(N // RF,),
    in_specs=[
        pl.BlockSpec((RF, 1), lambda i: (i, 0)),
        pl.BlockSpec((RF, C), lambda i: (i, 0)),
    ],
    out_specs=pl.BlockSpec((RF, C), lambda i: (i, 0)),
    out_shape=jax.ShapeDtypeStruct((N, C), jnp.float32),
)


def kernel(x, edge_index, W, b):
    ei = jnp.concatenate([edge_index, jnp.asarray(_EPAD)], axis=1)
    tar_r = ei[0].reshape(NS, NB, K)
    src_r = ei[1].reshape(NS, NB, K)

    degp = _deg_kernel(tar_r)
    g2, dcol = _g_call(degp, degp, x, W, b.reshape(1, C))
    acc = _agg_kernel(src_r, tar_r, g2)
    return _fin_call(dcol, acc)
